# baseline (speedup 1.0000x reference)
import functools

import jax
import jax.numpy as jnp
from jax import lax
from jax.experimental import pallas as pl
from jax.experimental.pallas import tpu as pltpu
from jax.experimental.pallas import tpu_sc as plsc

NUM_CAT = 3
CAT_VOCAB = 100000
NUM_HASH = 3
HASH_BINS = 100000
NUM_CONT_EMB = 2
CONT_BINS = 1000
NUM_CONT_NORM = 2
EMB = 32
BATCH = 16384
OUT_COLS = NUM_CAT * EMB + NUM_HASH * EMB + NUM_CONT_EMB * EMB + NUM_CONT_NORM

_info = plsc.get_sparse_core_info()
NC = _info.num_cores
NS = _info.num_subcores
L = _info.num_lanes
NW = NC * NS
CHUNK = BATCH // NW
NSLOT = 4
PLEN = 8 * 16
BROW = 2


def _deinterleave(stag2, ci, num_feat, lane):
    for f in range(num_feat):
        f_vec = jnp.full((L,), f, jnp.int32)

        def body(j, _, f=f, f_vec=f_vec):
            rows = j * L + lane
            v = plsc.load_gather(stag2, [rows, f_vec])
            ci[pl.ds(f * CHUNK + j * L, L)] = v
            return 0

        lax.fori_loop(0, CHUNK // L, body, 0)


def _tower_body(cat_idx, hash_idx, cont_vals, norm_vals,
                cat_tab, hash_tab, cont_tab, bnd, params,
                out,
                stag_i, stag_f, ci_cat, ci_hash, ci_cont, bnd_v, pv,
                rb0, rb1, rb2, rb3, normb, gsems, osems):
    rb = [rb0, rb1, rb2, rb3]
    wid = lax.axis_index("s") * NC + lax.axis_index("c")
    base = wid * CHUNK
    lane = lax.iota(jnp.int32, L)

    pltpu.sync_copy(cat_idx.at[pl.ds(base, CHUNK)], stag_i)
    _deinterleave(stag_i, ci_cat, NUM_CAT, lane)
    g = [None] * 8
    o = [None] * 9
    for f in range(NUM_CAT):
        g[f] = pltpu.async_copy(
            cat_tab.at[f].at[ci_cat.at[pl.ds(f * CHUNK, CHUNK)]], rb[f],
            gsems.at[f])

    pltpu.sync_copy(hash_idx.at[pl.ds(base, CHUNK)], stag_i)
    _deinterleave(stag_i, ci_hash, NUM_HASH, lane)
    g[3] = pltpu.async_copy(
        hash_tab.at[0].at[ci_hash.at[pl.ds(0, CHUNK)]], rb[3], gsems.at[3])

    pltpu.sync_copy(bnd, bnd_v.at[pl.ds(BROW, NUM_CONT_EMB)])
    pltpu.sync_copy(params, pv)
    pltpu.sync_copy(cont_vals.at[pl.ds(base, CHUNK)], stag_f)
    neg_big = jnp.full((L,), -jnp.inf, jnp.float32)
    pos_big = jnp.full((L,), jnp.inf, jnp.float32)
    for f in range(NUM_CONT_EMB):
        f_vec = jnp.full((L,), f, jnp.int32)
        fr_vec = jnp.full((L,), BROW + f, jnp.int32)
        b0 = pv[pl.ds((4 + f) * L, L)]
        blast = pv[pl.ds((6 + f) * L, L)]
        ist = (CONT_BINS - 1.0) / (blast - b0)

        def body(j, _, f=f, f_vec=f_vec, fr_vec=fr_vec, b0=b0, ist=ist):
            rows = j * L + lane
            x = plsc.load_gather(stag_f, [rows, f_vec])
            est = (x - b0) * ist
            est = jnp.minimum(jnp.maximum(est, -2.0), float(CONT_BINS + 1))
            c = est.astype(jnp.int32) + 1
            c = jnp.minimum(jnp.maximum(c, 0), CONT_BINS)
            for _ in range(2):
                lo = plsc.load_gather(bnd_v,
                                      [fr_vec, jnp.maximum(c - 1, 0)])
                lo = jnp.where(c == 0, neg_big, lo)
                hi = plsc.load_gather(
                    bnd_v, [fr_vec, jnp.minimum(c, CONT_BINS - 1)])
                hi = jnp.where(c == CONT_BINS, pos_big, hi)
                c = (c - (x < lo).astype(jnp.int32)
                     + (x >= hi).astype(jnp.int32))
            ci_cont[pl.ds(f * CHUNK + j * L, L)] = c
            return 0

        lax.fori_loop(0, CHUNK // L, body, 0)

    tail = [(hash_tab, ci_hash, 1), (hash_tab, ci_hash, 2),
            (cont_tab, ci_cont, 0), (cont_tab, ci_cont, 1)]
    for k in range(8):
        slot = k % NSLOT
        g[k].wait()
        o[k] = pltpu.async_copy(
            rb[slot],
            out.at[pl.ds(base, CHUNK), pl.ds(k * EMB, EMB)],
            osems.at[slot])
        if k + NSLOT < 8:
            o[k].wait()
            tab, cidx, f = tail[k]
            g[k + NSLOT] = pltpu.async_copy(
                tab.at[f].at[cidx.at[pl.ds(f * CHUNK, CHUNK)]],
                rb[slot], gsems.at[slot])
        if k == 3:
            pltpu.sync_copy(norm_vals.at[pl.ds(base, CHUNK)], stag_f)
            for f in range(NUM_CONT_NORM):
                f_vec = jnp.full((L,), f, jnp.int32)
                mv = pv[pl.ds(f * L, L)]
                sv = pv[pl.ds((2 + f) * L, L)]

                def nbody(j, _, f=f, f_vec=f_vec, mv=mv, sv=sv):
                    rows = j * L + lane
                    x = plsc.load_gather(stag_f, [rows, f_vec])
                    plsc.store_scatter(normb, [rows, f_vec], (x - mv) / sv)
                    return 0

                lax.fori_loop(0, CHUNK // L, nbody, 0)
            o[8] = pltpu.async_copy(
                normb,
                out.at[pl.ds(base, CHUNK),
                       pl.ds((NUM_CAT + NUM_HASH + NUM_CONT_EMB) * EMB,
                             NUM_CONT_NORM)],
                osems.at[NSLOT])
    for k in range(4, 9):
        o[k].wait()


_tower = functools.partial(
    pl.kernel,
    mesh=plsc.VectorSubcoreMesh(core_axis_name="c", subcore_axis_name="s"),
    out_type=jax.ShapeDtypeStruct((BATCH, OUT_COLS), jnp.float32),
    scratch_types=[
        pltpu.VMEM((CHUNK, NUM_CAT), jnp.int32),
        pltpu.VMEM((CHUNK, NUM_CONT_EMB), jnp.float32),
        pltpu.VMEM((NUM_CAT * CHUNK,), jnp.int32),
        pltpu.VMEM((NUM_HASH * CHUNK,), jnp.int32),
        pltpu.VMEM((NUM_CONT_EMB * CHUNK,), jnp.int32),
        pltpu.VMEM((BROW + NUM_CONT_EMB, CONT_BINS), jnp.float32),
        pltpu.VMEM((PLEN,), jnp.float32),
        pltpu.VMEM((CHUNK, EMB), jnp.float32),
        pltpu.VMEM((CHUNK, EMB), jnp.float32),
        pltpu.VMEM((CHUNK, EMB), jnp.float32),
        pltpu.VMEM((CHUNK, EMB), jnp.float32),
        pltpu.VMEM((CHUNK, NUM_CONT_NORM), jnp.float32),
        pltpu.SemaphoreType.DMA((NSLOT,)),
        pltpu.SemaphoreType.DMA((NSLOT + 1,)),
    ],
    compiler_params=pltpu.CompilerParams(
        use_tc_tiling_on_sc=False, needs_layout_passes=False),
)(_tower_body)


def kernel(cat_idx, hash_idx, cont_embed_vals, cont_norm_vals, cat_tables,
           hash_tables, cont_tables, cont_boundaries, norm_mean, norm_std):
    params = jnp.repeat(
        jnp.concatenate(
            [norm_mean.astype(jnp.float32), norm_std.astype(jnp.float32),
             cont_boundaries[:, 0], cont_boundaries[:, -1]]), L)
    return _tower(cat_idx, hash_idx, cont_embed_vals, cont_norm_vals,
                  cat_tables, hash_tables, cont_tables, cont_boundaries,
                  params)


# speedup vs baseline: 1.1656x; 1.1656x over previous
import functools

import jax
import jax.numpy as jnp
from jax import lax
from jax.experimental import pallas as pl
from jax.experimental.pallas import tpu as pltpu
from jax.experimental.pallas import tpu_sc as plsc

NUM_CAT = 3
CAT_VOCAB = 100000
NUM_HASH = 3
HASH_BINS = 100000
NUM_CONT_EMB = 2
CONT_BINS = 1000
NUM_CONT_NORM = 2
EMB = 32
BATCH = 16384
OUT_COLS = NUM_CAT * EMB + NUM_HASH * EMB + NUM_CONT_EMB * EMB + NUM_CONT_NORM

_info = plsc.get_sparse_core_info()
NC = _info.num_cores
NS = _info.num_subcores
L = _info.num_lanes
NW = NC * NS
CHUNK = BATCH // NW
NSLOT = 4
BPAD = 1024


def _tower_body(cat_idx_t, hash_idx_t, cont_vals_t, norm_flat,
                cat_tab, hash_tab, cont_tab, bpad, sparams,
                out,
                ci_cat, ci_hash, ci_cont, valsb, bndb, prmb, normb,
                rb0, rb1, rb2, rb3, gsems, osems):
    rb = [rb0, rb1, rb2, rb3]
    wid = lax.axis_index("s") * NC + lax.axis_index("c")
    base = wid * CHUNK
    lane = lax.iota(jnp.int32, L)

    g = [None] * 8
    o = [None] * 9
    for f in range(NUM_CAT):
        pltpu.sync_copy(cat_idx_t.at[f, pl.ds(base, CHUNK)],
                        ci_cat.at[pl.ds(f * CHUNK, CHUNK)])
        g[f] = pltpu.async_copy(
            cat_tab.at[f].at[ci_cat.at[pl.ds(f * CHUNK, CHUNK)]], rb[f],
            gsems.at[f])
    for f in range(NUM_HASH):
        pltpu.sync_copy(hash_idx_t.at[f, pl.ds(base, CHUNK)],
                        ci_hash.at[pl.ds(f * CHUNK, CHUNK)])
    g[3] = pltpu.async_copy(
        hash_tab.at[0].at[ci_hash.at[pl.ds(0, CHUNK)]], rb[3], gsems.at[3])

    pltpu.sync_copy(sparams, prmb)
    for f in range(NUM_CONT_EMB):
        pltpu.sync_copy(cont_vals_t.at[f, pl.ds(base, CHUNK)],
                        valsb.at[pl.ds(f * CHUNK, CHUNK)])
        pltpu.sync_copy(bpad.at[f], bndb.at[pl.ds(f * BPAD, BPAD)])
    neg_big = jnp.full((L,), -jnp.inf, jnp.float32)
    pos_big = jnp.full((L,), jnp.inf, jnp.float32)
    for f in range(NUM_CONT_EMB):
        fb = f * BPAD
        b0 = prmb[f]
        ist = prmb[2 + f]

        def body(j, _, f=f, fb=fb, b0=b0, ist=ist):
            x = valsb[pl.ds(f * CHUNK + j * L, L)]
            est = (x - b0) * ist
            est = jnp.minimum(jnp.maximum(est, -2.0), float(CONT_BINS + 1))
            c = est.astype(jnp.int32) + 1
            c = jnp.minimum(jnp.maximum(c, 0), CONT_BINS)
            for _ in range(2):
                lo = plsc.load_gather(bndb, [fb + c])
                lo = jnp.where(c == 0, neg_big, lo)
                hi = plsc.load_gather(bndb, [fb + c + 1])
                hi = jnp.where(c == CONT_BINS, pos_big, hi)
                c = (c - (x < lo).astype(jnp.int32)
                     + (x >= hi).astype(jnp.int32))
            ci_cont[pl.ds(f * CHUNK + j * L, L)] = c
            return 0

        lax.fori_loop(0, CHUNK // L, body, 0)

    tail = [(hash_tab, ci_hash, 1), (hash_tab, ci_hash, 2),
            (cont_tab, ci_cont, 0), (cont_tab, ci_cont, 1)]
    for k in range(8):
        slot = k % NSLOT
        g[k].wait()
        o[k] = pltpu.async_copy(
            rb[slot],
            out.at[pl.ds(base, CHUNK), pl.ds(k * EMB, EMB)],
            osems.at[slot])
        if k + NSLOT < 8:
            o[k].wait()
            tab, cidx, f = tail[k]
            g[k + NSLOT] = pltpu.async_copy(
                tab.at[f].at[cidx.at[pl.ds(f * CHUNK, CHUNK)]],
                rb[slot], gsems.at[slot])
        if k == 3:
            pltpu.sync_copy(norm_flat.at[pl.ds(base * NUM_CONT_NORM,
                                               CHUNK * NUM_CONT_NORM)],
                            valsb.at[pl.ds(0, CHUNK * NUM_CONT_NORM)])
            mean_pat = prmb[4]
            std_pat = prmb[5]
            col_idx = lane & 1
            for i in range(CHUNK * NUM_CONT_NORM // L):
                x = valsb[pl.ds(i * L, L)]
                y = (x - mean_pat) / std_pat
                row_idx = (lane + i * L) >> 1
                plsc.store_scatter(normb, [row_idx, col_idx], y)
            o[8] = pltpu.async_copy(
                normb,
                out.at[pl.ds(base, CHUNK),
                       pl.ds((NUM_CAT + NUM_HASH + NUM_CONT_EMB) * EMB,
                             NUM_CONT_NORM)],
                osems.at[NSLOT])
    for k in range(4, 9):
        o[k].wait()


_tower = functools.partial(
    pl.kernel,
    mesh=plsc.VectorSubcoreMesh(core_axis_name="c", subcore_axis_name="s"),
    out_type=jax.ShapeDtypeStruct((BATCH, OUT_COLS), jnp.float32),
    scratch_types=[
        pltpu.VMEM((NUM_CAT * CHUNK,), jnp.int32),
        pltpu.VMEM((NUM_HASH * CHUNK,), jnp.int32),
        pltpu.VMEM((NUM_CONT_EMB * CHUNK,), jnp.int32),
        pltpu.VMEM((NUM_CONT_EMB * CHUNK,), jnp.float32),
        pltpu.VMEM((NUM_CONT_EMB * BPAD,), jnp.float32),
        pltpu.VMEM((6, L), jnp.float32),
        pltpu.VMEM((CHUNK, NUM_CONT_NORM), jnp.float32),
        pltpu.VMEM((CHUNK, EMB), jnp.float32),
        pltpu.VMEM((CHUNK, EMB), jnp.float32),
        pltpu.VMEM((CHUNK, EMB), jnp.float32),
        pltpu.VMEM((CHUNK, EMB), jnp.float32),
        pltpu.SemaphoreType.DMA((NSLOT,)),
        pltpu.SemaphoreType.DMA((NSLOT + 1,)),
    ],
    compiler_params=pltpu.CompilerParams(
        use_tc_tiling_on_sc=False, needs_layout_passes=False),
)(_tower_body)


def kernel(cat_idx, hash_idx, cont_embed_vals, cont_norm_vals, cat_tables,
           hash_tables, cont_tables, cont_boundaries, norm_mean, norm_std):
    cat_idx_t = cat_idx.astype(jnp.int32).T
    hash_idx_t = hash_idx.astype(jnp.int32).T
    cont_vals_t = cont_embed_vals.T
    norm_flat = cont_norm_vals.reshape(-1)

    neg = jnp.full((NUM_CONT_EMB, 1), -jnp.inf, jnp.float32)
    pos = jnp.full((NUM_CONT_EMB, BPAD - CONT_BINS - 1), jnp.inf, jnp.float32)
    bpad = jnp.concatenate([neg, cont_boundaries, pos], axis=1)

    b0 = cont_boundaries[:, 0]
    inv_step = (CONT_BINS - 1) / (cont_boundaries[:, -1] - b0)
    sparams = jnp.stack([
        jnp.full((L,), b0[0], jnp.float32),
        jnp.full((L,), b0[1], jnp.float32),
        jnp.full((L,), inv_step[0], jnp.float32),
        jnp.full((L,), inv_step[1], jnp.float32),
        jnp.tile(norm_mean.astype(jnp.float32), L // NUM_CONT_NORM),
        jnp.tile(norm_std.astype(jnp.float32), L // NUM_CONT_NORM),
    ])

    return _tower(cat_idx_t, hash_idx_t, cont_vals_t, norm_flat,
                  cat_tables, hash_tables, cont_tables, bpad, sparams)


# speedup vs baseline: 1.1681x; 1.0021x over previous
import functools

import jax
import jax.numpy as jnp
from jax import lax
from jax.experimental import pallas as pl
from jax.experimental.pallas import tpu as pltpu
from jax.experimental.pallas import tpu_sc as plsc

NUM_CAT = 3
CAT_VOCAB = 100000
NUM_HASH = 3
HASH_BINS = 100000
NUM_CONT_EMB = 2
CONT_BINS = 1000
NUM_CONT_NORM = 2
EMB = 32
BATCH = 16384
OUT_COLS = NUM_CAT * EMB + NUM_HASH * EMB + NUM_CONT_EMB * EMB + NUM_CONT_NORM

_info = plsc.get_sparse_core_info()
NC = _info.num_cores
NS = _info.num_subcores
L = _info.num_lanes
NW = NC * NS
CHUNK = BATCH // NW
NSLOT = 6
BPAD = 1024


def _tower_body(cat_idx_t, hash_idx_t, cont_vals_t, norm_flat,
                cat_tab, hash_tab, cont_tab, bpad, sparams,
                out,
                ci_cat, ci_hash, ci_cont, valsb, bndb, prmb, normb,
                rb0, rb1, rb2, rb3, rb4, rb5, gsems, osems):
    rb = [rb0, rb1, rb2, rb3, rb4, rb5]
    wid = lax.axis_index("s") * NC + lax.axis_index("c")
    base = wid * CHUNK
    lane = lax.iota(jnp.int32, L)

    g = [None] * 8
    o = [None] * 9
    for f in range(NUM_CAT):
        pltpu.sync_copy(cat_idx_t.at[f, pl.ds(base, CHUNK)],
                        ci_cat.at[pl.ds(f * CHUNK, CHUNK)])
        g[f] = pltpu.async_copy(
            cat_tab.at[f].at[ci_cat.at[pl.ds(f * CHUNK, CHUNK)]], rb[f],
            gsems.at[f])
    for f in range(NUM_HASH):
        pltpu.sync_copy(hash_idx_t.at[f, pl.ds(base, CHUNK)],
                        ci_hash.at[pl.ds(f * CHUNK, CHUNK)])
        g[NUM_CAT + f] = pltpu.async_copy(
            hash_tab.at[f].at[ci_hash.at[pl.ds(f * CHUNK, CHUNK)]],
            rb[NUM_CAT + f], gsems.at[NUM_CAT + f])

    pltpu.sync_copy(sparams, prmb)
    for f in range(NUM_CONT_EMB):
        pltpu.sync_copy(cont_vals_t.at[f, pl.ds(base, CHUNK)],
                        valsb.at[pl.ds(f * CHUNK, CHUNK)])
        pltpu.sync_copy(bpad.at[f], bndb.at[pl.ds(f * BPAD, BPAD)])
    neg_big = jnp.full((L,), -jnp.inf, jnp.float32)
    pos_big = jnp.full((L,), jnp.inf, jnp.float32)
    for f in range(NUM_CONT_EMB):
        fb = f * BPAD
        b0 = prmb[f]
        ist = prmb[2 + f]

        def body(j, _, f=f, fb=fb, b0=b0, ist=ist):
            x = valsb[pl.ds(f * CHUNK + j * L, L)]
            est = (x - b0) * ist
            est = jnp.minimum(jnp.maximum(est, -2.0), float(CONT_BINS + 1))
            c = est.astype(jnp.int32) + 1
            c = jnp.minimum(jnp.maximum(c, 0), CONT_BINS)
            for _ in range(2):
                lo = plsc.load_gather(bndb, [fb + c])
                lo = jnp.where(c == 0, neg_big, lo)
                hi = plsc.load_gather(bndb, [fb + c + 1])
                hi = jnp.where(c == CONT_BINS, pos_big, hi)
                c = (c - (x < lo).astype(jnp.int32)
                     + (x >= hi).astype(jnp.int32))
            ci_cont[pl.ds(f * CHUNK + j * L, L)] = c
            return 0

        lax.fori_loop(0, CHUNK // L, body, 0)

    tail = [(cont_tab, ci_cont, 0), (cont_tab, ci_cont, 1)]
    for k in range(8):
        slot = k % NSLOT
        g[k].wait()
        o[k] = pltpu.async_copy(
            rb[slot],
            out.at[pl.ds(base, CHUNK), pl.ds(k * EMB, EMB)],
            osems.at[slot])
        if k + NSLOT < 8:
            o[k].wait()
            tab, cidx, f = tail[k]
            g[k + NSLOT] = pltpu.async_copy(
                tab.at[f].at[cidx.at[pl.ds(f * CHUNK, CHUNK)]],
                rb[slot], gsems.at[slot])
        if k == 3:
            pltpu.sync_copy(norm_flat.at[pl.ds(base * NUM_CONT_NORM,
                                               CHUNK * NUM_CONT_NORM)],
                            valsb.at[pl.ds(0, CHUNK * NUM_CONT_NORM)])
            mean_pat = prmb[4]
            std_pat = prmb[5]
            col_idx = lane & 1
            for i in range(CHUNK * NUM_CONT_NORM // L):
                x = valsb[pl.ds(i * L, L)]
                y = (x - mean_pat) / std_pat
                row_idx = (lane + i * L) >> 1
                plsc.store_scatter(normb, [row_idx, col_idx], y)
            o[8] = pltpu.async_copy(
                normb,
                out.at[pl.ds(base, CHUNK),
                       pl.ds((NUM_CAT + NUM_HASH + NUM_CONT_EMB) * EMB,
                             NUM_CONT_NORM)],
                osems.at[NSLOT])
    for k in range(4, 9):
        o[k].wait()


_tower = functools.partial(
    pl.kernel,
    mesh=plsc.VectorSubcoreMesh(core_axis_name="c", subcore_axis_name="s"),
    out_type=jax.ShapeDtypeStruct((BATCH, OUT_COLS), jnp.float32),
    scratch_types=[
        pltpu.VMEM((NUM_CAT * CHUNK,), jnp.int32),
        pltpu.VMEM((NUM_HASH * CHUNK,), jnp.int32),
        pltpu.VMEM((NUM_CONT_EMB * CHUNK,), jnp.int32),
        pltpu.VMEM((NUM_CONT_EMB * CHUNK,), jnp.float32),
        pltpu.VMEM((NUM_CONT_EMB * BPAD,), jnp.float32),
        pltpu.VMEM((6, L), jnp.float32),
        pltpu.VMEM((CHUNK, NUM_CONT_NORM), jnp.float32),
        pltpu.VMEM((CHUNK, EMB), jnp.float32),
        pltpu.VMEM((CHUNK, EMB), jnp.float32),
        pltpu.VMEM((CHUNK, EMB), jnp.float32),
        pltpu.VMEM((CHUNK, EMB), jnp.float32),
        pltpu.VMEM((CHUNK, EMB), jnp.float32),
        pltpu.VMEM((CHUNK, EMB), jnp.float32),
        pltpu.SemaphoreType.DMA((NSLOT,)),
        pltpu.SemaphoreType.DMA((NSLOT + 1,)),
    ],
    compiler_params=pltpu.CompilerParams(
        use_tc_tiling_on_sc=False, needs_layout_passes=False),
)(_tower_body)


def kernel(cat_idx, hash_idx, cont_embed_vals, cont_norm_vals, cat_tables,
           hash_tables, cont_tables, cont_boundaries, norm_mean, norm_std):
    cat_idx_t = cat_idx.astype(jnp.int32).T
    hash_idx_t = hash_idx.astype(jnp.int32).T
    cont_vals_t = cont_embed_vals.T
    norm_flat = cont_norm_vals.reshape(-1)

    neg = jnp.full((NUM_CONT_EMB, 1), -jnp.inf, jnp.float32)
    pos = jnp.full((NUM_CONT_EMB, BPAD - CONT_BINS - 1), jnp.inf, jnp.float32)
    bpad = jnp.concatenate([neg, cont_boundaries, pos], axis=1)

    b0 = cont_boundaries[:, 0]
    inv_step = (CONT_BINS - 1) / (cont_boundaries[:, -1] - b0)
    sparams = jnp.stack([
        jnp.full((L,), b0[0], jnp.float32),
        jnp.full((L,), b0[1], jnp.float32),
        jnp.full((L,), inv_step[0], jnp.float32),
        jnp.full((L,), inv_step[1], jnp.float32),
        jnp.tile(norm_mean.astype(jnp.float32), L // NUM_CONT_NORM),
        jnp.tile(norm_std.astype(jnp.float32), L // NUM_CONT_NORM),
    ])

    return _tower(cat_idx_t, hash_idx_t, cont_vals_t, norm_flat,
                  cat_tables, hash_tables, cont_tables, bpad, sparams)
